# Initial kernel scaffold; baseline (speedup 1.0000x reference)
import jax
import jax.numpy as jnp
from jax.experimental import pallas as pl


def kernel(x, pan, delta_p, delta_t):
    raise NotImplementedError("write your pallas kernel here")


# speedup vs baseline: 5.9133x; 5.9133x over previous
import functools

import jax
import jax.numpy as jnp
from jax import lax
from jax.experimental import pallas as pl
from jax.experimental.pallas import tpu as pltpu
from jax.experimental.pallas import tpu_sc as plsc

B, C, H, W = 2, 96, 224, 224
L = H * W
BL = B * L
NC, NS = 2, 16
NW = NC * NS
JPW = BL // NW
CH = 64
NCHUNK = JPW // CH
NV = C // 16
CP = 128


def _one(m):
    return jnp.where(m, jnp.float32(1.0), jnp.float32(0.0))


def _sc_body(xr, pr, sidx, dpx, dpy, out,
             p_v, pg_v, i00, i10, i01, i11, wbuf, dpx_v, dpy_v,
             xb0, xb1, xb2, xb3, pb0, pb1, pb2, pb3, ob, sem):
    cid = lax.axis_index("c")
    sid = lax.axis_index("s")
    wid = sid * NC + cid
    base = wid * JPW
    b_off = (base // L) * L

    def chunk(t, carry):
        j0 = base + t * CH
        pltpu.sync_copy(sidx.at[pl.ds(j0, CH)], p_v)
        for s in range(CH // 16):
            sl = pl.ds(s * 16, 16)
            pg_v[sl] = p_v[sl] + b_off
        c1 = pltpu.async_copy(dpx.at[pg_v], dpx_v, sem)
        c2 = pltpu.async_copy(dpy.at[pg_v], dpy_v, sem)
        c1.wait()
        c2.wait()
        for s in range(CH // 16):
            sl = pl.ds(s * 16, 16)
            p = p_v[sl]
            py = ((p.astype(jnp.float32) + 0.5)
                  * jnp.float32(1.0 / W)).astype(jnp.int32)
            px = p - py * W
            gx = px.astype(jnp.float32) * jnp.float32(2.0 / (W - 1)) - 1.0
            gy = py.astype(jnp.float32) * jnp.float32(2.0 / (H - 1)) - 1.0
            ix = ((gx + dpx_v[sl] + 1.0) * W - 1.0) * 0.5
            iy = ((gy + dpy_v[sl] + 1.0) * H - 1.0) * 0.5
            x0 = ix.astype(jnp.int32)
            x0f = x0.astype(jnp.float32)
            ax = x0f > ix
            x0 = x0 - jnp.where(ax, jnp.int32(1), jnp.int32(0))
            x0f = x0f - _one(ax)
            y0 = iy.astype(jnp.int32)
            y0f = y0.astype(jnp.float32)
            ay = y0f > iy
            y0 = y0 - jnp.where(ay, jnp.int32(1), jnp.int32(0))
            y0f = y0f - _one(ay)
            wx1 = ix - x0f
            wx0 = 1.0 - wx1
            wy1 = iy - y0f
            wy0 = 1.0 - wy1
            x1 = x0 + 1
            y1 = y0 + 1
            mx0 = _one((x0 >= 0) & (x0 <= W - 1))
            mx1 = _one((x1 >= 0) & (x1 <= W - 1))
            my0 = _one((y0 >= 0) & (y0 <= H - 1))
            my1 = _one((y1 >= 0) & (y1 <= H - 1))
            xc0 = jnp.clip(x0, 0, W - 1)
            xc1 = jnp.clip(x1, 0, W - 1)
            r0 = jnp.clip(y0, 0, H - 1) * W + b_off
            r1 = jnp.clip(y1, 0, H - 1) * W + b_off
            i00[sl] = r0 + xc0
            i10[sl] = r0 + xc1
            i01[sl] = r1 + xc0
            i11[sl] = r1 + xc1
            wbuf[0, sl] = wx0 * wy0 * mx0 * my0
            wbuf[1, sl] = wx1 * wy0 * mx1 * my0
            wbuf[2, sl] = wx0 * wy1 * mx0 * my1
            wbuf[3, sl] = wx1 * wy1 * mx1 * my1
        cs = [
            pltpu.async_copy(xr.at[i00], xb0, sem),
            pltpu.async_copy(xr.at[i10], xb1, sem),
            pltpu.async_copy(xr.at[i01], xb2, sem),
            pltpu.async_copy(xr.at[i11], xb3, sem),
            pltpu.async_copy(pr.at[i00], pb0, sem),
            pltpu.async_copy(pr.at[i10], pb1, sem),
            pltpu.async_copy(pr.at[i01], pb2, sem),
            pltpu.async_copy(pr.at[i11], pb3, sem),
        ]
        for cc in cs:
            cc.wait()

        def gbody(j, gc):
            wv0 = wbuf[0, pl.ds(j, 16)]
            wv1 = wbuf[1, pl.ds(j, 16)]
            wv2 = wbuf[2, pl.ds(j, 16)]
            wv3 = wbuf[3, pl.ds(j, 16)]
            w0 = wv0[0]
            w1 = wv1[0]
            w2 = wv2[0]
            w3 = wv3[0]
            for v in range(NV):
                slc = pl.ds(v * 16, 16)
                ob[2 * j, slc] = (xb0[j, slc] * w0 + xb1[j, slc] * w1
                                  + xb2[j, slc] * w2 + xb3[j, slc] * w3)
                ob[2 * j + 1, slc] = (pb0[j, slc] * w0 + pb1[j, slc] * w1
                                      + pb2[j, slc] * w2 + pb3[j, slc] * w3)
            return gc

        lax.fori_loop(0, CH, gbody, 0)
        pltpu.sync_copy(ob, out.at[pl.ds(2 * j0, 2 * CH)])
        return carry

    lax.fori_loop(0, NCHUNK, chunk, 0)


_MESH = plsc.VectorSubcoreMesh(core_axis_name="c", subcore_axis_name="s")

_sc_call = functools.partial(
    pl.kernel,
    mesh=_MESH,
    out_type=jax.ShapeDtypeStruct((2 * BL, C), jnp.float32),
    scratch_types=[
        pltpu.VMEM((CH,), jnp.int32),
        pltpu.VMEM((CH,), jnp.int32),
        pltpu.VMEM((CH,), jnp.int32),
        pltpu.VMEM((CH,), jnp.int32),
        pltpu.VMEM((CH,), jnp.int32),
        pltpu.VMEM((CH,), jnp.int32),
        pltpu.VMEM((4, CH + 16), jnp.float32),
        pltpu.VMEM((CH,), jnp.float32),
        pltpu.VMEM((CH,), jnp.float32),
        pltpu.VMEM((CH, CP), jnp.float32),
        pltpu.VMEM((CH, CP), jnp.float32),
        pltpu.VMEM((CH, CP), jnp.float32),
        pltpu.VMEM((CH, CP), jnp.float32),
        pltpu.VMEM((CH, CP), jnp.float32),
        pltpu.VMEM((CH, CP), jnp.float32),
        pltpu.VMEM((CH, CP), jnp.float32),
        pltpu.VMEM((CH, CP), jnp.float32),
        pltpu.VMEM((2 * CH, C), jnp.float32),
        pltpu.SemaphoreType.DMA,
    ],
)(_sc_body)


def kernel(x, pan, delta_p, delta_t):
    b, c, h, w = x.shape
    l = h * w
    xr = jnp.transpose(x, (0, 2, 3, 1)).reshape(b * l, c)
    pr = jnp.transpose(pan, (0, 2, 3, 1)).reshape(b * l, c)
    xr = jnp.pad(xr, ((0, 0), (0, CP - c)))
    pr = jnp.pad(pr, ((0, 0), (0, CP - c)))
    dpx = delta_p[:, 0].reshape(b * l)
    dpy = delta_p[:, 1].reshape(b * l)
    ref_idx = (jnp.arange(l, dtype=jnp.float32).reshape(1, 1, h, w)
               / (l - 1)) * 2.0 - 1.0
    keys = (ref_idx + delta_t).reshape(b, l)
    sidx = jnp.argsort(keys, axis=1).astype(jnp.int32).reshape(b * l)
    out = _sc_call(xr, pr, sidx, dpx, dpy)
    return out.reshape(b, 2 * l, c)


# speedup vs baseline: 6.2837x; 1.0626x over previous
import functools

import jax
import jax.numpy as jnp
from jax import lax
from jax.experimental import pallas as pl
from jax.experimental.pallas import tpu as pltpu
from jax.experimental.pallas import tpu_sc as plsc

B, C, H, W = 2, 96, 224, 224
L = H * W
BL = B * L
NC, NS = 2, 16
NW = NC * NS
JPW = BL // NW
CH = 32
NCHUNK = JPW // CH
NV = C // 16
CP = 128


def _one(m):
    return jnp.where(m, jnp.float32(1.0), jnp.float32(0.0))


def _sc_body(xr, pr, sidx, dpx, dpy, out,
             p_v, pg_v, dpx_v, dpy_v,
             idx0, idx1, wb0, wb1, gb0, gb1, ob, semd, sem0, sem1):
    cid = lax.axis_index("c")
    sid = lax.axis_index("s")
    wid = sid * NC + cid
    base = wid * JPW
    b_off = (base // L) * L

    idxs = (idx0, idx1)
    wbs = (wb0, wb1)
    gbs = (gb0, gb1)
    sems = (sem0, sem1)

    def prepare(t, s):
        j0 = base + t * CH
        idx = idxs[s]
        wbuf = wbs[s]
        gb = gbs[s]
        sem = sems[s]
        pltpu.sync_copy(sidx.at[pl.ds(j0, CH)], p_v)
        for g in range(CH // 16):
            sl = pl.ds(g * 16, 16)
            pg_v[sl] = p_v[sl] + b_off
        c1 = pltpu.async_copy(dpx.at[pg_v], dpx_v, semd)
        c2 = pltpu.async_copy(dpy.at[pg_v], dpy_v, semd)
        c1.wait()
        c2.wait()
        for g in range(CH // 16):
            sl = pl.ds(g * 16, 16)
            p = p_v[sl]
            py = ((p.astype(jnp.float32) + 0.5)
                  * jnp.float32(1.0 / W)).astype(jnp.int32)
            px = p - py * W
            gx = px.astype(jnp.float32) * jnp.float32(2.0 / (W - 1)) - 1.0
            gy = py.astype(jnp.float32) * jnp.float32(2.0 / (H - 1)) - 1.0
            ix = ((gx + dpx_v[sl] + 1.0) * W - 1.0) * 0.5
            iy = ((gy + dpy_v[sl] + 1.0) * H - 1.0) * 0.5
            x0 = ix.astype(jnp.int32)
            x0f = x0.astype(jnp.float32)
            ax = x0f > ix
            x0 = x0 - jnp.where(ax, jnp.int32(1), jnp.int32(0))
            x0f = x0f - _one(ax)
            y0 = iy.astype(jnp.int32)
            y0f = y0.astype(jnp.float32)
            ay = y0f > iy
            y0 = y0 - jnp.where(ay, jnp.int32(1), jnp.int32(0))
            y0f = y0f - _one(ay)
            wx1 = ix - x0f
            wx0 = 1.0 - wx1
            wy1 = iy - y0f
            wy0 = 1.0 - wy1
            x1 = x0 + 1
            y1 = y0 + 1
            mx0 = _one((x0 >= 0) & (x0 <= W - 1))
            mx1 = _one((x1 >= 0) & (x1 <= W - 1))
            my0 = _one((y0 >= 0) & (y0 <= H - 1))
            my1 = _one((y1 >= 0) & (y1 <= H - 1))
            xc0 = jnp.clip(x0, 0, W - 1)
            xc1 = jnp.clip(x1, 0, W - 1)
            r0 = jnp.clip(y0, 0, H - 1) * W + b_off
            r1 = jnp.clip(y1, 0, H - 1) * W + b_off
            idx[0, sl] = r0 + xc0
            idx[1, sl] = r0 + xc1
            idx[2, sl] = r1 + xc0
            idx[3, sl] = r1 + xc1
            wbuf[0, sl] = wx0 * wy0 * mx0 * my0
            wbuf[1, sl] = wx1 * wy0 * mx1 * my0
            wbuf[2, sl] = wx0 * wy1 * mx0 * my1
            wbuf[3, sl] = wx1 * wy1 * mx1 * my1
        for k in range(4):
            pltpu.async_copy(xr.at[idx.at[k]], gb.at[k], sem)
            pltpu.async_copy(pr.at[idx.at[k]], gb.at[4 + k], sem)

    def combine(t, s):
        j0 = base + t * CH
        wbuf = wbs[s]
        gb = gbs[s]
        sem = sems[s]
        for k in range(8):
            pltpu.make_async_copy(xr.at[pl.ds(0, CH)], gb.at[k], sem).wait()

        def gbody(j, gc):
            wv0 = wbuf[0, pl.ds(j, 16)]
            wv1 = wbuf[1, pl.ds(j, 16)]
            wv2 = wbuf[2, pl.ds(j, 16)]
            wv3 = wbuf[3, pl.ds(j, 16)]
            w0 = wv0[0]
            w1 = wv1[0]
            w2 = wv2[0]
            w3 = wv3[0]
            for v in range(NV):
                slc = pl.ds(v * 16, 16)
                ob[2 * j, slc] = (gb[0, j, slc] * w0 + gb[1, j, slc] * w1
                                  + gb[2, j, slc] * w2 + gb[3, j, slc] * w3)
                ob[2 * j + 1, slc] = (gb[4, j, slc] * w0 + gb[5, j, slc] * w1
                                      + gb[6, j, slc] * w2 + gb[7, j, slc] * w3)
            return gc

        lax.fori_loop(0, CH, gbody, 0)
        pltpu.sync_copy(ob, out.at[pl.ds(2 * j0, 2 * CH)])

    prepare(0, 0)

    def pair(u, carry):
        prepare(2 * u + 1, 1)
        combine(2 * u, 0)

        @pl.when(u < NCHUNK // 2 - 1)
        def _():
            prepare(2 * u + 2, 0)

        combine(2 * u + 1, 1)
        return carry

    lax.fori_loop(0, NCHUNK // 2, pair, 0)


_MESH = plsc.VectorSubcoreMesh(core_axis_name="c", subcore_axis_name="s")

_sc_call = functools.partial(
    pl.kernel,
    mesh=_MESH,
    out_type=jax.ShapeDtypeStruct((2 * BL, C), jnp.float32),
    scratch_types=[
        pltpu.VMEM((CH,), jnp.int32),
        pltpu.VMEM((CH,), jnp.int32),
        pltpu.VMEM((CH,), jnp.float32),
        pltpu.VMEM((CH,), jnp.float32),
        pltpu.VMEM((4, CH), jnp.int32),
        pltpu.VMEM((4, CH), jnp.int32),
        pltpu.VMEM((4, CH + 16), jnp.float32),
        pltpu.VMEM((4, CH + 16), jnp.float32),
        pltpu.VMEM((8, CH, CP), jnp.float32),
        pltpu.VMEM((8, CH, CP), jnp.float32),
        pltpu.VMEM((2 * CH, C), jnp.float32),
        pltpu.SemaphoreType.DMA,
        pltpu.SemaphoreType.DMA,
        pltpu.SemaphoreType.DMA,
    ],
)(_sc_body)


def kernel(x, pan, delta_p, delta_t):
    b, c, h, w = x.shape
    l = h * w
    xr = jnp.transpose(x, (0, 2, 3, 1)).reshape(b * l, c)
    pr = jnp.transpose(pan, (0, 2, 3, 1)).reshape(b * l, c)
    xr = jnp.pad(xr, ((0, 0), (0, CP - c)))
    pr = jnp.pad(pr, ((0, 0), (0, CP - c)))
    dpx = delta_p[:, 0].reshape(b * l)
    dpy = delta_p[:, 1].reshape(b * l)
    ref_idx = (jnp.arange(l, dtype=jnp.float32).reshape(1, 1, h, w)
               / (l - 1)) * 2.0 - 1.0
    keys = (ref_idx + delta_t).reshape(b, l)
    sidx = jnp.argsort(keys, axis=1).astype(jnp.int32).reshape(b * l)
    out = _sc_call(xr, pr, sidx, dpx, dpy)
    return out.reshape(b, 2 * l, c)
